# baseline (speedup 1.0000x reference)
import functools

import jax
import jax.numpy as jnp
from jax.experimental import pallas as pl

_BM = 400


def _xw_kernel(x_ref, w_ref, o_ref):
    o_ref[...] = jnp.dot(
        x_ref[...], w_ref[...],
        preferred_element_type=jnp.float32,
        precision=jax.lax.Precision.HIGHEST,
    )


def _layer_kernel(adj_ref, v_ref, b_ref, w2_ref, o_ref, *, last):
    acc = jnp.dot(
        adj_ref[...].astype(jnp.bfloat16),
        v_ref[...].astype(jnp.bfloat16),
        preferred_element_type=jnp.float32,
    )
    h = acc + b_ref[...]
    h = jnp.where(h >= 0, h, 0.01 * h)
    if last:
        o_ref[...] = h
    else:
        o_ref[...] = jnp.dot(
            h, w2_ref[...],
            preferred_element_type=jnp.float32,
            precision=jax.lax.Precision.HIGHEST,
        )


def _layer(adj, v, b, w2, *, last):
    n = adj.shape[0]
    grid = (n // _BM,)
    return pl.pallas_call(
        functools.partial(_layer_kernel, last=last),
        grid=grid,
        in_specs=[
            pl.BlockSpec((_BM, n), lambda i: (i, 0)),
            pl.BlockSpec(v.shape, lambda i: (0, 0)),
            pl.BlockSpec(b.shape, lambda i: (0, 0)),
            pl.BlockSpec(w2.shape, lambda i: (0, 0)),
        ],
        out_specs=pl.BlockSpec((_BM, v.shape[1]), lambda i: (i, 0)),
        out_shape=jax.ShapeDtypeStruct((n, v.shape[1]), jnp.float32),
    )(adj, v, b, w2)


def kernel(x, adj, W1, b1, W2, b2):
    xw = pl.pallas_call(
        _xw_kernel,
        out_shape=jax.ShapeDtypeStruct((x.shape[0], W1.shape[1]), jnp.float32),
    )(x, W1)
    b1r = b1.reshape(1, -1)
    b2r = b2.reshape(1, -1)
    h1w = _layer(adj, xw, b1r, W2, last=False)
    out = _layer(adj, h1w, b2r, W2, last=True)
    return out


# speedup vs baseline: 1.0123x; 1.0123x over previous
import functools

import jax
import jax.numpy as jnp
from jax.experimental import pallas as pl

_BM = 400


def _xw_kernel(x_ref, w_ref, o_ref):
    o_ref[...] = jnp.dot(
        x_ref[...], w_ref[...],
        preferred_element_type=jnp.float32,
        precision=jax.lax.Precision.HIGHEST,
    ).astype(jnp.bfloat16)


def _layer_kernel(adj_ref, v_ref, b_ref, w2_ref, o_ref, *, last):
    acc = jnp.dot(
        adj_ref[...].astype(jnp.bfloat16),
        v_ref[...],
        preferred_element_type=jnp.float32,
    )
    h = acc + b_ref[...]
    h = jnp.where(h >= 0, h, 0.01 * h)
    if last:
        o_ref[...] = h
    else:
        o_ref[...] = jnp.dot(
            h, w2_ref[...],
            preferred_element_type=jnp.float32,
            precision=jax.lax.Precision.HIGHEST,
        ).astype(jnp.bfloat16)


def _layer(adj, v, b, w2, *, last):
    n = adj.shape[0]
    grid = (n // _BM,)
    return pl.pallas_call(
        functools.partial(_layer_kernel, last=last),
        grid=grid,
        in_specs=[
            pl.BlockSpec((_BM, n), lambda i: (i, 0)),
            pl.BlockSpec(v.shape, lambda i: (0, 0)),
            pl.BlockSpec(b.shape, lambda i: (0, 0)),
            pl.BlockSpec(w2.shape, lambda i: (0, 0)),
        ],
        out_specs=pl.BlockSpec((_BM, v.shape[1]), lambda i: (i, 0)),
        out_shape=jax.ShapeDtypeStruct(
            (n, v.shape[1]), jnp.float32 if last else jnp.bfloat16
        ),
    )(adj, v, b, w2)


def kernel(x, adj, W1, b1, W2, b2):
    xw = pl.pallas_call(
        _xw_kernel,
        out_shape=jax.ShapeDtypeStruct((x.shape[0], W1.shape[1]), jnp.bfloat16),
    )(x, W1)
    b1r = b1.reshape(1, -1)
    b2r = b2.reshape(1, -1)
    h1w = _layer(adj, xw, b1r, W2, last=False)
    out = _layer(adj, h1w, b2r, W2, last=True)
    return out
